# Initial kernel scaffold; baseline (speedup 1.0000x reference)
import jax
import jax.numpy as jnp
from jax.experimental import pallas as pl


def kernel(x, edge_index, W1, b1, W2, b2, W3, b3):
    raise NotImplementedError("write your pallas kernel here")


# speedup vs baseline: 3.5651x; 3.5651x over previous
import functools

import jax
import jax.numpy as jnp
from jax import lax
from jax.experimental import pallas as pl
from jax.experimental.pallas import tpu as pltpu
from jax.experimental.pallas import tpu_sc as plsc

N = 10000
E = 320000
D_IN = 128
D_H = 128
D_OUT = 64

NC = 2
NS = 16
NTILES = NC * NS
EPT = E // NTILES
CHUNK = 80
NCHUNK = EPT // CHUNK
NP = 10240
RPT = NP // NS
WB = 128

_MESH = plsc.VectorSubcoreMesh(
    core_axis_name="c", subcore_axis_name="s", num_cores=NC, num_subcores=NS
)

BN = 1000
GRID = N // BN


EPT2 = E // NS
NCHUNK2 = EPT2 // CHUNK
WBN = RPT // WB


@functools.partial(
    pl.kernel,
    out_type=jax.ShapeDtypeStruct((NC, NP, D_H), jnp.float32),
    mesh=_MESH,
    scratch_types=[
        pltpu.VMEM((CHUNK,), jnp.int32),
        pltpu.VMEM((CHUNK, D_H), jnp.float32),
        pltpu.VMEM((WB, D_H), jnp.float32),
        pltpu.VMEM_SHARED((NP, D_H), jnp.float32),
    ],
)
def _degree_kernel(ef_hbm, out_hbm, sidx, ones_v, zb, acc_sh):
    c = lax.axis_index("c")
    s = lax.axis_index("s")
    zero = jnp.zeros((16,), jnp.float32)
    one = jnp.ones((16,), jnp.float32)

    @pl.loop(0, WB)
    def _(i):
        for j in range(D_H // 16):
            zb[i, pl.ds(j * 16, 16)] = zero

    @pl.loop(0, CHUNK)
    def _(i):
        for j in range(D_H // 16):
            ones_v[i, pl.ds(j * 16, 16)] = one

    r0 = s * RPT
    for k in range(WBN):
        pltpu.sync_copy(zb, acc_sh.at[pl.ds(r0 + k * WB, WB)])
    plsc.subcore_barrier()

    base = c * E + s * EPT2

    @pl.loop(0, NCHUNK2)
    def _(ch):
        off = base + ch * CHUNK
        pltpu.sync_copy(ef_hbm.at[pl.ds(off, CHUNK)], sidx)
        pltpu.sync_copy(ones_v, acc_sh.at[sidx], add=True)

    plsc.subcore_barrier()
    for k in range(WBN):
        pltpu.sync_copy(acc_sh.at[pl.ds(r0 + k * WB, WB)], zb)
        pltpu.sync_copy(zb, out_hbm.at[c, pl.ds(r0 + k * WB, WB)])


def _make_agg(d):
    @functools.partial(
        pl.kernel,
        out_type=jax.ShapeDtypeStruct((NC, NP, d), jnp.float32),
        mesh=_MESH,
        scratch_types=[
            pltpu.VMEM((CHUNK,), jnp.int32),
            pltpu.VMEM((CHUNK,), jnp.int32),
            pltpu.VMEM((CHUNK, d), jnp.float32),
            pltpu.VMEM((WB, d), jnp.float32),
            pltpu.VMEM_SHARED((NP, d), jnp.float32),
            pltpu.SemaphoreType.DMA,
        ],
    )
    def _agg(g_hbm, src_hbm, dst_hbm, out_hbm, sidx, didx, rows_v, wb_v,
             acc_sh, sem):
        c = lax.axis_index("c")
        s = lax.axis_index("s")
        tid = s * NC + c
        zero = jnp.zeros((16,), jnp.float32)

        @pl.loop(0, WB)
        def _(i):
            for j in range(d // 16):
                wb_v[i, pl.ds(j * 16, 16)] = zero

        r0 = s * RPT
        for k in range(RPT // WB):
            pltpu.sync_copy(wb_v, acc_sh.at[pl.ds(r0 + k * WB, WB)])
        plsc.subcore_barrier()

        base = tid * EPT

        @pl.loop(0, NCHUNK)
        def _(ch):
            off = base + ch * CHUNK
            pltpu.sync_copy(src_hbm.at[pl.ds(off, CHUNK)], sidx)
            pltpu.sync_copy(dst_hbm.at[pl.ds(off, CHUNK)], didx)
            pltpu.async_copy(g_hbm.at[sidx], rows_v, sem).wait()
            pltpu.sync_copy(rows_v, acc_sh.at[didx], add=True)

        plsc.subcore_barrier()
        for k in range(RPT // WB):
            pltpu.sync_copy(acc_sh.at[pl.ds(r0 + k * WB, WB)], wb_v)
            pltpu.sync_copy(wb_v, out_hbm.at[c, pl.ds(r0 + k * WB, WB)])

    return _agg


_agg128 = _make_agg(D_H)


def _norm_body(degp_ref, iso_ref, isi_ref):
    dsrc = degp_ref[0]
    ddst = degp_ref[1]
    iso_ref[...] = lax.rsqrt(
        jnp.maximum(jnp.max(dsrc, axis=1, keepdims=True), 1.0))
    isi_ref[...] = lax.rsqrt(
        jnp.maximum(jnp.max(ddst, axis=1, keepdims=True), 1.0))


_norm = pl.pallas_call(
    _norm_body,
    grid=(GRID,),
    in_specs=[pl.BlockSpec((NC, BN, D_H), lambda i: (0, i, 0))],
    out_specs=[
        pl.BlockSpec((BN, 1), lambda i: (i, 0)),
        pl.BlockSpec((BN, 1), lambda i: (i, 0)),
    ],
    out_shape=[
        jax.ShapeDtypeStruct((N, 1), jnp.float32),
        jax.ShapeDtypeStruct((N, 1), jnp.float32),
    ],
)


def _mm_scale_body(h_ref, w_ref, iso_ref, o_ref):
    o_ref[...] = jnp.dot(
        h_ref[...], w_ref[...], preferred_element_type=jnp.float32
    ) * iso_ref[...]


def _make_mm_scale(din, dout):
    return pl.pallas_call(
        _mm_scale_body,
        grid=(GRID,),
        in_specs=[
            pl.BlockSpec((BN, din), lambda i: (i, 0)),
            pl.BlockSpec((din, dout), lambda i: (0, 0)),
            pl.BlockSpec((BN, 1), lambda i: (i, 0)),
        ],
        out_specs=pl.BlockSpec((BN, dout), lambda i: (i, 0)),
        out_shape=jax.ShapeDtypeStruct((N, dout), jnp.float32),
    )


def _mid_body(p_ref, isi_ref, b_ref, w_ref, iso_ref, o_ref):
    h = jnp.maximum(
        (p_ref[0] + p_ref[1]) * isi_ref[...] + b_ref[...], 0.0)
    o_ref[...] = jnp.dot(
        h, w_ref[...], preferred_element_type=jnp.float32) * iso_ref[...]


def _make_mid(din, dout):
    return pl.pallas_call(
        _mid_body,
        grid=(GRID,),
        in_specs=[
            pl.BlockSpec((NC, BN, din), lambda i: (0, i, 0)),
            pl.BlockSpec((BN, 1), lambda i: (i, 0)),
            pl.BlockSpec((1, din), lambda i: (0, 0)),
            pl.BlockSpec((din, dout), lambda i: (0, 0)),
            pl.BlockSpec((BN, 1), lambda i: (i, 0)),
        ],
        out_specs=pl.BlockSpec((BN, dout), lambda i: (i, 0)),
        out_shape=jax.ShapeDtypeStruct((N, dout), jnp.float32),
    )


def _final_body(p_ref, isi_ref, b_ref, o_ref):
    s = p_ref[0, :, :D_OUT] + p_ref[1, :, :D_OUT]
    o_ref[...] = s * isi_ref[...] + b_ref[...]


_final = pl.pallas_call(
    _final_body,
    grid=(GRID,),
    in_specs=[
        pl.BlockSpec((NC, BN, D_H), lambda i: (0, i, 0)),
        pl.BlockSpec((BN, 1), lambda i: (i, 0)),
        pl.BlockSpec((1, D_OUT), lambda i: (0, 0)),
    ],
    out_specs=pl.BlockSpec((BN, D_OUT), lambda i: (i, 0)),
    out_shape=jax.ShapeDtypeStruct((N, D_OUT), jnp.float32),
)

_mm1 = _make_mm_scale(D_IN, D_H)
_mid2 = _make_mid(D_H, D_H)
_mid3 = _make_mid(D_H, D_H)


def kernel(x, edge_index, W1, b1, W2, b2, W3, b3):
    src = edge_index[0]
    dst = edge_index[1]
    W3p = jnp.pad(W3, ((0, 0), (0, D_H - D_OUT)))
    edge_flat = jnp.concatenate([src, dst])
    degp = _degree_kernel(edge_flat)
    iso, isi = _norm(degp)
    g1 = _mm1(x, W1, iso)
    p1 = _agg128(g1, src, dst)
    g2 = _mid2(p1, isi, b1.reshape(1, D_H), W2, iso)
    p2 = _agg128(g2, src, dst)
    g3 = _mid3(p2, isi, b2.reshape(1, D_H), W3p, iso)
    p3 = _agg128(g3, src, dst)
    return _final(p3, isi, b3.reshape(1, D_OUT))
